# baseline (speedup 1.0000x reference)
import jax
import jax.numpy as jnp
from jax.experimental import pallas as pl
from jax.experimental.pallas import tpu as pltpu

D_MODEL = 1024
D_FF = 4096
E = 8
RT = 256
FT = 512
N1 = D_MODEL // RT
N2 = D_FF // FT


def _body(x_hbm, wg_hbm, w1_hbm, b1_hbm, w2_hbm, b2_hbm, o_ref,
          x_ref, wg_ref, w1_buf, w2_buf, b1_buf, b2_buf,
          semx, sem1, sem2, semb):
    cx = pltpu.make_async_copy(x_hbm, x_ref, semx.at[0])
    cwg = pltpu.make_async_copy(wg_hbm, wg_ref, semx.at[1])
    cx.start()
    cwg.start()
    cx.wait()
    cwg.wait()
    logits = jax.lax.dot_general(
        x_ref[...], wg_ref[...], (((1,), (1,)), ((), ())),
        preferred_element_type=jnp.float32)
    e = jnp.argmax(logits, axis=1)[0].astype(jnp.int32)

    cb1 = pltpu.make_async_copy(b1_hbm.at[pl.ds(e, 1), :], b1_buf, semb.at[0])
    cb2 = pltpu.make_async_copy(b2_hbm.at[pl.ds(e, 1), :], b2_buf, semb.at[1])

    def cp1(r):
        return pltpu.make_async_copy(
            w1_hbm.at[e, pl.ds(r * RT, RT), :], w1_buf.at[r], sem1.at[r])

    def cp2(k):
        return pltpu.make_async_copy(
            w2_hbm.at[e, pl.ds(k * FT, FT), :], w2_buf.at[k], sem2.at[k])

    cb1.start()
    cb2.start()
    for r in range(N1):
        cp1(r).start()
    for k in range(N2):
        cp2(k).start()

    cb1.wait()
    h = b1_buf[...]
    for r in range(N1):
        cp1(r).wait()
        h = h + jnp.dot(x_ref[:, r * RT:(r + 1) * RT], w1_buf[r],
                        preferred_element_type=jnp.float32)
    h = jax.nn.gelu(h)
    cb2.wait()
    acc = b2_buf[...]
    for k in range(N2):
        cp2(k).wait()
        acc = acc + jnp.dot(h[:, k * FT:(k + 1) * FT], w2_buf[k],
                            preferred_element_type=jnp.float32)
    o_ref[...] = acc


def kernel(x, Wg, W1, b1, W2, b2):
    return pl.pallas_call(
        _body,
        in_specs=[
            pl.BlockSpec(memory_space=pltpu.MemorySpace.HBM),
            pl.BlockSpec(memory_space=pltpu.MemorySpace.HBM),
            pl.BlockSpec(memory_space=pltpu.MemorySpace.HBM),
            pl.BlockSpec(memory_space=pltpu.MemorySpace.HBM),
            pl.BlockSpec(memory_space=pltpu.MemorySpace.HBM),
            pl.BlockSpec(memory_space=pltpu.MemorySpace.HBM),
        ],
        out_specs=pl.BlockSpec(memory_space=pltpu.MemorySpace.VMEM),
        out_shape=jax.ShapeDtypeStruct((1, D_MODEL), jnp.float32),
        scratch_shapes=[
            pltpu.VMEM((1, D_MODEL), jnp.float32),
            pltpu.VMEM((E, D_MODEL), jnp.float32),
            pltpu.VMEM((N1, RT, D_FF), jnp.float32),
            pltpu.VMEM((N2, FT, D_MODEL), jnp.float32),
            pltpu.VMEM((1, D_FF), jnp.float32),
            pltpu.VMEM((1, D_MODEL), jnp.float32),
            pltpu.SemaphoreType.DMA((2,)),
            pltpu.SemaphoreType.DMA((N1,)),
            pltpu.SemaphoreType.DMA((N2,)),
            pltpu.SemaphoreType.DMA((2,)),
        ],
    )(x, Wg.T, W1, b1, W2, b2)
